# baseline (speedup 1.0000x reference)
import jax
import jax.numpy as jnp
from jax import lax
from jax.experimental import pallas as pl
from jax.experimental.pallas import tpu as pltpu
from jax.experimental.pallas import tpu_sc as plsc

_N = 10000
_E = 320000
_DIN = 128
_DHID = 64
_DOUT = 4

_NC = 2
_NS = 16
_NW = _NC * _NS
_EPW = _E // _NW
_C = 80
_NCHUNK = _EPW // _C
_B = 5
_NP = 10240
_RPT = _NP // _NS
_EP = 320
_F2 = 16

_mesh = plsc.VectorSubcoreMesh(core_axis_name="c", subcore_axis_name="s")
_sc_params = pltpu.CompilerParams(use_tc_tiling_on_sc=False)


def _stage_and_loop(y_hbm, sidx, didx, msg, acc, gsem, ssem, F,
                    extra_scatter=None):
  for b in range(_B):
    pltpu.async_copy(y_hbm.at[sidx.at[b]], msg.at[b], gsem[b])

  def drain_scatter(pb, j):
    pltpu.make_async_copy(msg.at[pb], acc.at[didx.at[j]], ssem[pb]).wait()
    if extra_scatter is not None:
      extra_scatter.drain(pb, j)

  def group(g, carry):
    for b in range(_B):
      j = g * _B + b
      pltpu.make_async_copy(y_hbm.at[sidx.at[j]], msg.at[b], gsem[b]).wait()
      pltpu.async_copy(msg.at[b], acc.at[didx.at[j]], ssem[b], add=True)
      if extra_scatter is not None:
        extra_scatter.issue(b, j)
      pb = (b - 1) % _B

      @pl.when(j >= 1)
      def _():
        drain_scatter(pb, j)

        @pl.when(j - 1 + _B < _NCHUNK)
        def _():
          pltpu.async_copy(y_hbm.at[sidx.at[j - 1 + _B]], msg.at[pb],
                           gsem[pb])
    return carry

  lax.fori_loop(0, _NCHUNK // _B, group, 0)
  drain_scatter(_B - 1, _NCHUNK - 1)


class _CountScatter:
  def __init__(self, onesv, cacc, didx, didx2, ssem):
    self.onesv, self.cacc, self.didx, self.didx2, self.ssem = (
        onesv, cacc, didx, didx2, ssem)

  def issue(self, b, j):
    pltpu.async_copy(self.onesv, self.cacc.at[self.didx.at[j]],
                     self.ssem[b], add=True)
    pltpu.async_copy(self.onesv, self.cacc.at[self.didx2.at[j]],
                     self.ssem[b], add=True)

  def drain(self, pb, j):
    pltpu.make_async_copy(self.onesv, self.cacc.at[self.didx.at[j]],
                          self.ssem[pb]).wait()
    pltpu.make_async_copy(self.onesv, self.cacc.at[self.didx2.at[j]],
                          self.ssem[pb]).wait()


def _make_pass1():
  out_type = (jax.ShapeDtypeStruct((_NC, _NP, _DHID), jnp.float32),
              jax.ShapeDtypeStruct((_NP,), jnp.float32))
  scratch = [
      pltpu.VMEM((_NCHUNK, _C), jnp.int32),
      pltpu.VMEM((_NCHUNK, _C), jnp.int32),
      pltpu.VMEM((_NCHUNK, _C), jnp.int32),
      pltpu.VMEM((_B, _C, _DHID), jnp.float32),
      pltpu.VMEM((_C,), jnp.float32),
      pltpu.VMEM((_EP, _DHID), jnp.float32),
      pltpu.VMEM((_EP,), jnp.float32),
      pltpu.VMEM((_EP,), jnp.float32),
      pltpu.VMEM_SHARED((_NP, _DHID), jnp.float32),
      pltpu.VMEM_SHARED((_NP,), jnp.float32),
  ] + [pltpu.SemaphoreType.DMA] * (2 * _B)

  def body(y_hbm, ei_hbm, zf_hbm, zc_hbm, ones_hbm, out_hbm, rcnt_hbm,
           sidx, didx, didx2, msg, onesv, eb, cb, rb, acc, cacc, *sems):
    gsem, ssem = sems[:_B], sems[_B:]
    c = lax.axis_index("c")
    s = lax.axis_index("s")
    wid = c * _NS + s
    wid2 = (1 - c) * _NS + s
    rows = pl.ds(s * _RPT, _RPT)
    pltpu.sync_copy(zf_hbm.at[rows], acc.at[rows])
    pltpu.sync_copy(zc_hbm.at[rows], cacc.at[rows])
    pltpu.sync_copy(ei_hbm.at[0].at[wid], sidx)
    pltpu.sync_copy(ei_hbm.at[1].at[wid], didx)
    pltpu.sync_copy(ei_hbm.at[1].at[wid2], didx2)
    pltpu.sync_copy(ones_hbm, onesv)
    plsc.subcore_barrier()

    cs = _CountScatter(onesv, cacc, didx, didx2, ssem)
    _stage_and_loop(y_hbm, sidx, didx, msg, acc, gsem, ssem, _DHID,
                    extra_scatter=cs)
    plsc.subcore_barrier()

    for half in range(2):
      rbase = s * _RPT + half * _EP
      strip = pl.ds(rbase, _EP)
      pltpu.sync_copy(acc.at[strip], eb)
      pltpu.sync_copy(cacc.at[strip], cb)

      for k in range(_EP // 16):
        lanes = pl.ds(16 * k, 16)
        rb[lanes] = 1.0 / jnp.maximum(cb[lanes], 1.0)

      def egroup(g, carry):
        cvec = rb[pl.ds(g * 16, 16)]
        for l in range(16):
          rr = g * 16 + l
          rv = cvec[l]
          for k in range(_DHID // 16):
            col = pl.ds(16 * k, 16)
            eb[rr, col] = eb[rr, col] * rv
        return carry

      lax.fori_loop(0, _EP // 16, egroup, 0)
      pltpu.sync_copy(eb, out_hbm.at[c].at[strip])

      @pl.when(c == 0)
      def _():
        pltpu.sync_copy(rb, rcnt_hbm.at[strip])

  return pl.kernel(body, mesh=_mesh, out_type=out_type,
                   scratch_types=scratch, compiler_params=_sc_params)


def _make_pass2():
  out_type = jax.ShapeDtypeStruct((_NC, _NP, _F2), jnp.float32)
  scratch = [
      pltpu.VMEM((_NCHUNK, _C), jnp.int32),
      pltpu.VMEM((_NCHUNK, _C), jnp.int32),
      pltpu.VMEM((_B, _C, _F2), jnp.float32),
      pltpu.VMEM((_RPT,), jnp.float32),
      pltpu.VMEM((_EP, _F2), jnp.float32),
      pltpu.VMEM_SHARED((_NP, _F2), jnp.float32),
  ] + [pltpu.SemaphoreType.DMA] * (2 * _B)

  def body(y_hbm, ei_hbm, zf_hbm, rcnt_hbm, out_hbm,
           sidx, didx, msg, rstage, eb, acc, *sems):
    gsem, ssem = sems[:_B], sems[_B:]
    c = lax.axis_index("c")
    s = lax.axis_index("s")
    wid = c * _NS + s
    rows = pl.ds(s * _RPT, _RPT)
    pltpu.sync_copy(zf_hbm.at[rows], acc.at[rows])
    pltpu.sync_copy(ei_hbm.at[0].at[wid], sidx)
    pltpu.sync_copy(ei_hbm.at[1].at[wid], didx)
    pltpu.sync_copy(rcnt_hbm.at[rows], rstage)
    plsc.subcore_barrier()

    _stage_and_loop(y_hbm, sidx, didx, msg, acc, gsem, ssem, _F2)
    plsc.subcore_barrier()

    for half in range(2):
      strip = pl.ds(s * _RPT + half * _EP, _EP)
      pltpu.sync_copy(acc.at[strip], eb)

      def egroup(g, carry):
        cvec = rstage[pl.ds(half * _EP + g * 16, 16)]
        for l in range(16):
          rr = g * 16 + l
          eb[rr] = eb[rr] * cvec[l]
        return carry

      lax.fori_loop(0, _EP // 16, egroup, 0)
      pltpu.sync_copy(eb, out_hbm.at[c].at[strip])

  return pl.kernel(body, mesh=_mesh, out_type=out_type,
                   scratch_types=scratch, compiler_params=_sc_params)


_pass1 = _make_pass1()
_pass2 = _make_pass2()


def _mm_body(x_ref, w_ref, y_ref):
  y_ref[...] = jnp.dot(x_ref[...], w_ref[...],
                       preferred_element_type=jnp.float32)


def _make_mm(n, k, m):
  return pl.pallas_call(
      _mm_body, out_shape=jax.ShapeDtypeStruct((n, m), jnp.float32))


_mm_y1 = _make_mm(_N, _DIN, _DHID)
_mm_xr = _make_mm(_N, _DIN, _DHID)
_mm_hr = _make_mm(_N, _DHID, _F2)


def _mid_body(s1p_ref, xr_ref, b1_ref, w2_ref, y2_ref, h_ref):
  sp = s1p_ref[...]
  o = sp[0, :_N] + sp[1, :_N] + b1_ref[...] + xr_ref[...]
  nrm = jnp.sqrt(jnp.sum(o * o, axis=-1, keepdims=True))
  o = o / jnp.maximum(nrm, 1e-12)
  h = jnp.maximum(o, 0.0)
  h_ref[...] = h
  y2_ref[...] = jnp.dot(h, w2_ref[...], preferred_element_type=jnp.float32)


_mid = pl.pallas_call(
    _mid_body,
    out_shape=(jax.ShapeDtypeStruct((_N, _F2), jnp.float32),
               jax.ShapeDtypeStruct((_N, _DHID), jnp.float32)),
)


def _fin_body(s2p_ref, hr2_ref, b2_ref, o_ref):
  sp = s2p_ref[...]
  o = sp[0, :_N] + sp[1, :_N] + b2_ref[...] + hr2_ref[...]
  nrm = jnp.sqrt(jnp.sum(o * o, axis=-1, keepdims=True))
  o = o / jnp.maximum(nrm, 1e-12)
  o_ref[...] = o[:, :_DOUT]


_fin = pl.pallas_call(
    _fin_body,
    out_shape=jax.ShapeDtypeStruct((_N, _DOUT), jnp.float32),
)


def kernel(x, edge_index, W1l, b1l, W1r, W2l, b2l, W2r):
  ei = edge_index.reshape(2, _NW, _NCHUNK, _C)
  w2l = jnp.zeros((_DHID, _F2), jnp.float32).at[:, :_DOUT].set(W2l.T)
  w2r = jnp.zeros((_DHID, _F2), jnp.float32).at[:, :_DOUT].set(W2r.T)
  zf1 = jnp.zeros((_NP, _DHID), jnp.float32)
  zc = jnp.zeros((_NP,), jnp.float32)
  zf2 = jnp.zeros((_NP, _F2), jnp.float32)
  onesb = jnp.ones((_C,), jnp.float32)
  b1 = b1l.reshape(1, _DHID)
  b2 = jnp.zeros((1, _F2), jnp.float32).at[0, :_DOUT].set(b2l)

  y1 = _mm_y1(x, W1l.T)
  s1p, rcnt = _pass1(y1, ei, zf1, zc, onesb)
  xr1 = _mm_xr(x, W1r.T)
  y2p, h = _mid(s1p, xr1, b1, w2l)
  s2p = _pass2(y2p, ei, zf2, rcnt)
  hr2 = _mm_hr(h, w2r)
  return _fin(s2p, hr2, b2)


# speedup vs baseline: 1.0623x; 1.0623x over previous
import jax
import jax.numpy as jnp
from jax import lax
from jax.experimental import pallas as pl
from jax.experimental.pallas import tpu as pltpu
from jax.experimental.pallas import tpu_sc as plsc

_N = 10000
_E = 320000
_DIN = 128
_DHID = 64
_DOUT = 4

_NC = 2
_NS = 16
_NW = _NC * _NS
_EPW = _E // _NW
_C = 80
_NCHUNK = _EPW // _C
_B = 5
_NP = 10240
_RPT = _NP // _NS
_EP = 320
_F2 = 16

_mesh = plsc.VectorSubcoreMesh(core_axis_name="c", subcore_axis_name="s")
_sc_params = pltpu.CompilerParams(use_tc_tiling_on_sc=False)


def _stage_and_loop(y_hbm, sidx, didx, msg, acc, gsem, ssem, F,
                    extra_scatter=None):
  for b in range(_B):
    pltpu.async_copy(y_hbm.at[sidx.at[b]], msg.at[b], gsem[b])

  def drain_scatter(pb, j):
    pltpu.make_async_copy(msg.at[pb], acc.at[didx.at[j]], ssem[pb]).wait()
    if extra_scatter is not None:
      extra_scatter.drain(pb, j)

  def group(g, carry):
    for b in range(_B):
      j = g * _B + b
      pltpu.make_async_copy(y_hbm.at[sidx.at[j]], msg.at[b], gsem[b]).wait()
      pltpu.async_copy(msg.at[b], acc.at[didx.at[j]], ssem[b], add=True)
      if extra_scatter is not None:
        extra_scatter.issue(b, j)
      pb = (b - 1) % _B

      @pl.when(j >= 1)
      def _():
        drain_scatter(pb, j)

        @pl.when(j - 1 + _B < _NCHUNK)
        def _():
          pltpu.async_copy(y_hbm.at[sidx.at[j - 1 + _B]], msg.at[pb],
                           gsem[pb])
    return carry

  lax.fori_loop(0, _NCHUNK // _B, group, 0)
  drain_scatter(_B - 1, _NCHUNK - 1)


class _CountScatter:
  def __init__(self, onesv, cacc, didx, didx2, ssem):
    self.onesv, self.cacc, self.didx, self.didx2, self.ssem = (
        onesv, cacc, didx, didx2, ssem)

  def issue(self, b, j):
    pltpu.async_copy(self.onesv, self.cacc.at[self.didx.at[j]],
                     self.ssem[b], add=True)
    pltpu.async_copy(self.onesv, self.cacc.at[self.didx2.at[j]],
                     self.ssem[b], add=True)

  def drain(self, pb, j):
    pltpu.make_async_copy(self.onesv, self.cacc.at[self.didx.at[j]],
                          self.ssem[pb]).wait()
    pltpu.make_async_copy(self.onesv, self.cacc.at[self.didx2.at[j]],
                          self.ssem[pb]).wait()


def _make_pass1():
  out_type = (jax.ShapeDtypeStruct((_NC, _NP, _DHID), jnp.float32),
              jax.ShapeDtypeStruct((_NP,), jnp.float32))
  scratch = [
      pltpu.VMEM((_NCHUNK, _C), jnp.int32),
      pltpu.VMEM((_NCHUNK, _C), jnp.int32),
      pltpu.VMEM((_NCHUNK, _C), jnp.int32),
      pltpu.VMEM((_B, _C, _DHID), jnp.float32),
      pltpu.VMEM((_C,), jnp.float32),
      pltpu.VMEM((_EP, _DHID), jnp.float32),
      pltpu.VMEM((_EP,), jnp.float32),
      pltpu.VMEM((_EP,), jnp.float32),
      pltpu.VMEM_SHARED((_NP, _DHID), jnp.float32),
      pltpu.VMEM_SHARED((_NP,), jnp.float32),
  ] + [pltpu.SemaphoreType.DMA] * (2 * _B)

  def body(y_hbm, ei_hbm, zf_hbm, zc_hbm, ones_hbm, out_hbm, rcnt_hbm,
           sidx, didx, didx2, msg, onesv, eb, cb, rb, acc, cacc, *sems):
    gsem, ssem = sems[:_B], sems[_B:]
    c = lax.axis_index("c")
    s = lax.axis_index("s")
    wid = c * _NS + s
    wid2 = (1 - c) * _NS + s
    rows = pl.ds(s * _RPT, _RPT)
    pltpu.sync_copy(zf_hbm, acc.at[rows])
    pltpu.sync_copy(zc_hbm, cacc.at[rows])
    pltpu.sync_copy(ei_hbm.at[0].at[wid], sidx)
    pltpu.sync_copy(ei_hbm.at[1].at[wid], didx)
    pltpu.sync_copy(ei_hbm.at[1].at[wid2], didx2)
    pltpu.sync_copy(ones_hbm, onesv)
    plsc.subcore_barrier()

    cs = _CountScatter(onesv, cacc, didx, didx2, ssem)
    _stage_and_loop(y_hbm, sidx, didx, msg, acc, gsem, ssem, _DHID,
                    extra_scatter=cs)
    plsc.subcore_barrier()

    for half in range(2):
      rbase = s * _RPT + half * _EP
      strip = pl.ds(rbase, _EP)
      pltpu.sync_copy(acc.at[strip], eb)
      pltpu.sync_copy(cacc.at[strip], cb)

      for k in range(_EP // 16):
        lanes = pl.ds(16 * k, 16)
        rb[lanes] = 1.0 / jnp.maximum(cb[lanes], 1.0)

      def egroup(g, carry):
        cvec = rb[pl.ds(g * 16, 16)]
        for l in range(16):
          rr = g * 16 + l
          rv = cvec[l]
          for k in range(_DHID // 16):
            col = pl.ds(16 * k, 16)
            eb[rr, col] = eb[rr, col] * rv
        return carry

      lax.fori_loop(0, _EP // 16, egroup, 0)
      pltpu.sync_copy(eb, out_hbm.at[c].at[strip])

      @pl.when(c == 0)
      def _():
        pltpu.sync_copy(rb, rcnt_hbm.at[strip])

  return pl.kernel(body, mesh=_mesh, out_type=out_type,
                   scratch_types=scratch, compiler_params=_sc_params)


def _make_pass2():
  out_type = jax.ShapeDtypeStruct((_NC, _NP, _F2), jnp.float32)
  scratch = [
      pltpu.VMEM((_NCHUNK, _C), jnp.int32),
      pltpu.VMEM((_NCHUNK, _C), jnp.int32),
      pltpu.VMEM((_B, _C, _F2), jnp.float32),
      pltpu.VMEM((_RPT,), jnp.float32),
      pltpu.VMEM((_EP, _F2), jnp.float32),
      pltpu.VMEM_SHARED((_NP, _F2), jnp.float32),
  ] + [pltpu.SemaphoreType.DMA] * (2 * _B)

  def body(y_hbm, ei_hbm, zf_hbm, rcnt_hbm, out_hbm,
           sidx, didx, msg, rstage, eb, acc, *sems):
    gsem, ssem = sems[:_B], sems[_B:]
    c = lax.axis_index("c")
    s = lax.axis_index("s")
    wid = c * _NS + s
    rows = pl.ds(s * _RPT, _RPT)
    pltpu.sync_copy(zf_hbm, acc.at[rows])
    pltpu.sync_copy(ei_hbm.at[0].at[wid], sidx)
    pltpu.sync_copy(ei_hbm.at[1].at[wid], didx)
    pltpu.sync_copy(rcnt_hbm.at[rows], rstage)
    plsc.subcore_barrier()

    _stage_and_loop(y_hbm, sidx, didx, msg, acc, gsem, ssem, _F2)
    plsc.subcore_barrier()

    for half in range(2):
      strip = pl.ds(s * _RPT + half * _EP, _EP)
      pltpu.sync_copy(acc.at[strip], eb)

      def egroup(g, carry):
        cvec = rstage[pl.ds(half * _EP + g * 16, 16)]
        for l in range(16):
          rr = g * 16 + l
          eb[rr] = eb[rr] * cvec[l]
        return carry

      lax.fori_loop(0, _EP // 16, egroup, 0)
      pltpu.sync_copy(eb, out_hbm.at[c].at[strip])

  return pl.kernel(body, mesh=_mesh, out_type=out_type,
                   scratch_types=scratch, compiler_params=_sc_params)


_pass1 = _make_pass1()
_pass2 = _make_pass2()


def _mm_body(x_ref, w_ref, y_ref):
  y_ref[...] = jnp.dot(x_ref[...], w_ref[...],
                       preferred_element_type=jnp.float32)


def _make_mm(n, k, m):
  return pl.pallas_call(
      _mm_body, out_shape=jax.ShapeDtypeStruct((n, m), jnp.float32))


_mm_y1 = _make_mm(_N, _DIN, _DHID)
_mm_xr = _make_mm(_N, _DIN, _DHID)
_mm_hr = _make_mm(_N * _DHID // 512, 512, 128)


def _mid_body(s1p_ref, xr_ref, b1_ref, w2_ref, y2_ref, h_ref):
  sp = s1p_ref[...]
  o = sp[0, :_N] + sp[1, :_N] + b1_ref[...] + xr_ref[...]
  nrm = jnp.sqrt(jnp.sum(o * o, axis=-1, keepdims=True))
  o = o / jnp.maximum(nrm, 1e-12)
  h = jnp.maximum(o, 0.0)
  h_ref[...] = h
  y2_ref[...] = jnp.dot(h, w2_ref[...], preferred_element_type=jnp.float32)


_mid = pl.pallas_call(
    _mid_body,
    out_shape=(jax.ShapeDtypeStruct((_N, _F2), jnp.float32),
               jax.ShapeDtypeStruct((_N, _DHID), jnp.float32)),
)


_NPK = _N * _F2 // 128


def _fin_body(s2p_ref, hrp_ref, b2p_ref, msk_ref, o_ref):
  sp = s2p_ref[...]
  o = sp[0, :_NPK] + sp[1, :_NPK] + b2p_ref[...] + hrp_ref[...]
  ss = jnp.dot(o * o, msk_ref[...], preferred_element_type=jnp.float32)
  o_ref[...] = o / jnp.maximum(jnp.sqrt(ss), 1e-12)


_fin = pl.pallas_call(
    _fin_body,
    out_shape=jax.ShapeDtypeStruct((_NPK, 128), jnp.float32),
)


def kernel(x, edge_index, W1l, b1l, W1r, W2l, b2l, W2r):
  ei = edge_index.reshape(2, _NW, _NCHUNK, _C)
  w2l = jnp.zeros((_DHID, _F2), jnp.float32).at[:, :_DOUT].set(W2l.T)
  w2r = jnp.zeros((_DHID, _F2), jnp.float32).at[:, :_DOUT].set(W2r.T)
  zf1 = jnp.zeros((_RPT, _DHID), jnp.float32)
  zc = jnp.zeros((_RPT,), jnp.float32)
  zf2 = jnp.zeros((_RPT, _F2), jnp.float32)
  onesb = jnp.ones((_C,), jnp.float32)
  b1 = b1l.reshape(1, _DHID)
  b2 = jnp.zeros((1, _F2), jnp.float32).at[0, :_DOUT].set(b2l)
  b2p = jnp.tile(b2, (1, 8))
  lane = jnp.arange(128) // _F2
  msk = (lane[:, None] == lane[None, :]).astype(jnp.float32)
  w2r_bd = jnp.zeros((512, 128), jnp.float32)
  for i in range(8):
    w2r_bd = w2r_bd.at[64 * i:64 * (i + 1), 16 * i:16 * (i + 1)].set(w2r)

  y1 = _mm_y1(x, W1l.T)
  s1p, rcnt = _pass1(y1, ei, zf1, zc, onesb)
  xr1 = _mm_xr(x, W1r.T)
  y2p, h = _mid(s1p, xr1, b1, w2l)
  s2p = _pass2(y2p, ei, zf2, rcnt)
  hp = h.reshape(_N * _DHID // 512, 512)
  hrp = _mm_hr(hp, w2r_bd)
  outp = _fin(s2p.reshape(_NC, _NP * _F2 // 128, 128), hrp, b2p, msk)
  return outp.reshape(_N, _F2)[:, :_DOUT]


# speedup vs baseline: 1.1671x; 1.0987x over previous
import jax
import jax.numpy as jnp
from jax import lax
from jax.experimental import pallas as pl
from jax.experimental.pallas import tpu as pltpu
from jax.experimental.pallas import tpu_sc as plsc

_N = 10000
_E = 320000
_DIN = 128
_DHID = 64
_DOUT = 4

_NC = 2
_NS = 16
_NW = _NC * _NS
_EPW = _E // _NW
_C = 80
_NCHUNK = _EPW // _C
_B = 5
_NP = 10240
_RPT = _NP // _NS
_EP = 320
_F2 = 16

_mesh = plsc.VectorSubcoreMesh(core_axis_name="c", subcore_axis_name="s")
_sc_params = pltpu.CompilerParams(use_tc_tiling_on_sc=False)


def _stage_and_loop(y_hbm, sidx, didx, msg, acc, gsem, ssem, F,
                    extra_scatter=None):
  for b in range(_B):
    pltpu.async_copy(y_hbm.at[sidx.at[b]], msg.at[b], gsem[b])

  def drain_scatter(pb, j):
    pltpu.make_async_copy(msg.at[pb], acc.at[didx.at[j]], ssem[pb]).wait()
    if extra_scatter is not None:
      extra_scatter.drain(pb, j)

  def group(g, carry):
    for b in range(_B):
      j = g * _B + b
      pltpu.make_async_copy(y_hbm.at[sidx.at[j]], msg.at[b], gsem[b]).wait()
      pltpu.async_copy(msg.at[b], acc.at[didx.at[j]], ssem[b], add=True)
      if extra_scatter is not None:
        extra_scatter.issue(b, j)
      pb = (b - 1) % _B

      @pl.when(j >= 1)
      def _():
        drain_scatter(pb, j)

        @pl.when(j - 1 + _B < _NCHUNK)
        def _():
          pltpu.async_copy(y_hbm.at[sidx.at[j - 1 + _B]], msg.at[pb],
                           gsem[pb])
    return carry

  lax.fori_loop(0, _NCHUNK // _B, group, 0)
  drain_scatter(_B - 1, _NCHUNK - 1)


class _CountScatter:
  def __init__(self, onesv, cacc, didx, didx2, ssem):
    self.onesv, self.cacc, self.didx, self.didx2, self.ssem = (
        onesv, cacc, didx, didx2, ssem)

  def issue(self, b, j):
    pltpu.async_copy(self.onesv, self.cacc.at[self.didx.at[j]],
                     self.ssem[b], add=True)
    pltpu.async_copy(self.onesv, self.cacc.at[self.didx2.at[j]],
                     self.ssem[b], add=True)

  def drain(self, pb, j):
    pltpu.make_async_copy(self.onesv, self.cacc.at[self.didx.at[j]],
                          self.ssem[pb]).wait()
    pltpu.make_async_copy(self.onesv, self.cacc.at[self.didx2.at[j]],
                          self.ssem[pb]).wait()


def _make_pass1():
  out_type = (jax.ShapeDtypeStruct((_NC, _NP, _DHID), jnp.float32),
              jax.ShapeDtypeStruct((_NP,), jnp.float32))
  scratch = [
      pltpu.VMEM((_NCHUNK, _C), jnp.int32),
      pltpu.VMEM((_NCHUNK, _C), jnp.int32),
      pltpu.VMEM((_NCHUNK, _C), jnp.int32),
      pltpu.VMEM((_B, _C, _DHID), jnp.float32),
      pltpu.VMEM((_C,), jnp.float32),
      pltpu.VMEM((_EP, _DHID), jnp.float32),
      pltpu.VMEM((_EP,), jnp.float32),
      pltpu.VMEM((_EP,), jnp.float32),
      pltpu.VMEM_SHARED((_NP, _DHID), jnp.float32),
      pltpu.VMEM_SHARED((_NP,), jnp.float32),
  ] + [pltpu.SemaphoreType.DMA] * (2 * _B)

  def body(y_hbm, ei_hbm, zf_hbm, zc_hbm, ones_hbm, out_hbm, rcnt_hbm,
           sidx, didx, didx2, msg, onesv, eb, cb, rb, acc, cacc, *sems):
    gsem, ssem = sems[:_B], sems[_B:]
    c = lax.axis_index("c")
    s = lax.axis_index("s")
    wid = c * _NS + s
    wid2 = (1 - c) * _NS + s
    rows = pl.ds(s * _RPT, _RPT)
    pltpu.sync_copy(zf_hbm, acc.at[rows])
    pltpu.sync_copy(zc_hbm, cacc.at[rows])
    pltpu.sync_copy(ei_hbm.at[0].at[wid], sidx)
    pltpu.sync_copy(ei_hbm.at[1].at[wid], didx)
    pltpu.sync_copy(ei_hbm.at[1].at[wid2], didx2)
    pltpu.sync_copy(ones_hbm, onesv)
    plsc.subcore_barrier()

    cs = _CountScatter(onesv, cacc, didx, didx2, ssem)
    _stage_and_loop(y_hbm, sidx, didx, msg, acc, gsem, ssem, _DHID,
                    extra_scatter=cs)
    plsc.subcore_barrier()

    for half in range(2):
      rbase = s * _RPT + half * _EP
      strip = pl.ds(rbase, _EP)
      pltpu.sync_copy(acc.at[strip], eb)
      pltpu.sync_copy(cacc.at[strip], cb)

      for k in range(_EP // 16):
        lanes = pl.ds(16 * k, 16)
        rb[lanes] = 1.0 / jnp.maximum(cb[lanes], 1.0)

      def egroup(g, carry):
        cvec = rb[pl.ds(g * 16, 16)]
        for l in range(16):
          rr = g * 16 + l
          rv = cvec[l]
          for k in range(_DHID // 16):
            col = pl.ds(16 * k, 16)
            eb[rr, col] = eb[rr, col] * rv
        return carry

      lax.fori_loop(0, _EP // 16, egroup, 0)
      pltpu.sync_copy(eb, out_hbm.at[c].at[strip])

      @pl.when(c == 0)
      def _():
        pltpu.sync_copy(rb, rcnt_hbm.at[strip])

  return pl.kernel(body, mesh=_mesh, out_type=out_type,
                   scratch_types=scratch, compiler_params=_sc_params)


def _make_pass2():
  out_type = jax.ShapeDtypeStruct((_NC, _NP, _F2), jnp.float32)
  scratch = [
      pltpu.VMEM((_NCHUNK, _C), jnp.int32),
      pltpu.VMEM((_NCHUNK, _C), jnp.int32),
      pltpu.VMEM((_B, _C, _F2), jnp.float32),
      pltpu.VMEM((_RPT,), jnp.float32),
      pltpu.VMEM((_EP, _F2), jnp.float32),
      pltpu.VMEM_SHARED((_NP, _F2), jnp.float32),
  ] + [pltpu.SemaphoreType.DMA] * (2 * _B)

  def body(y_hbm, ei_hbm, zf_hbm, rcnt_hbm, out_hbm,
           sidx, didx, msg, rstage, eb, acc, *sems):
    gsem, ssem = sems[:_B], sems[_B:]
    c = lax.axis_index("c")
    s = lax.axis_index("s")
    wid = c * _NS + s
    rows = pl.ds(s * _RPT, _RPT)
    pltpu.sync_copy(zf_hbm, acc.at[rows])
    pltpu.sync_copy(ei_hbm.at[0].at[wid], sidx)
    pltpu.sync_copy(ei_hbm.at[1].at[wid], didx)
    pltpu.sync_copy(rcnt_hbm.at[rows], rstage)
    plsc.subcore_barrier()

    _stage_and_loop(y_hbm, sidx, didx, msg, acc, gsem, ssem, _F2)
    plsc.subcore_barrier()

    for half in range(2):
      strip = pl.ds(s * _RPT + half * _EP, _EP)
      pltpu.sync_copy(acc.at[strip], eb)

      def egroup(g, carry):
        cvec = rstage[pl.ds(half * _EP + g * 16, 16)]
        for l in range(16):
          rr = g * 16 + l
          eb[rr] = eb[rr] * cvec[l]
        return carry

      lax.fori_loop(0, _EP // 16, egroup, 0)
      pltpu.sync_copy(eb, out_hbm.at[c].at[strip])

  return pl.kernel(body, mesh=_mesh, out_type=out_type,
                   scratch_types=scratch, compiler_params=_sc_params)


_pass1 = _make_pass1()
_pass2 = _make_pass2()


def _mm_body(x_ref, w_ref, y_ref):
  y_ref[...] = jnp.dot(x_ref[...], w_ref[...],
                       preferred_element_type=jnp.float32)


def _make_mm(n, k, m):
  return pl.pallas_call(
      _mm_body, out_shape=jax.ShapeDtypeStruct((n, m), jnp.float32))


_mm_y1 = _make_mm(_N, _DIN, _DHID)
_mm_xr = _make_mm(_N, _DIN, _DHID)
_mm_hr = _make_mm(_N * _DHID // 512, 512, 128)


_NPK1 = _N * _DHID // 128


def _mid_body(s1p_ref, xr_ref, b1_ref, msk_ref, w2_ref, y2_ref, h_ref):
  sp = s1p_ref[...]
  o = sp[0, :_NPK1] + sp[1, :_NPK1] + b1_ref[...] + xr_ref[...]
  ss = jnp.dot(o * o, msk_ref[...], preferred_element_type=jnp.float32)
  h = jnp.maximum(o / jnp.maximum(jnp.sqrt(ss), 1e-12), 0.0)
  h_ref[...] = h
  y2_ref[...] = jnp.dot(h, w2_ref[...], preferred_element_type=jnp.float32)


_mid = pl.pallas_call(
    _mid_body,
    out_shape=(jax.ShapeDtypeStruct((_NPK1, 32), jnp.float32),
               jax.ShapeDtypeStruct((_NPK1, 128), jnp.float32)),
)


_NPK = _N * _F2 // 128


def _fin_body(s2p_ref, hrp_ref, b2p_ref, msk_ref, o_ref):
  sp = s2p_ref[...]
  o = sp[0, :_NPK] + sp[1, :_NPK] + b2p_ref[...] + hrp_ref[...]
  ss = jnp.dot(o * o, msk_ref[...], preferred_element_type=jnp.float32)
  o_ref[...] = o / jnp.maximum(jnp.sqrt(ss), 1e-12)


_fin = pl.pallas_call(
    _fin_body,
    out_shape=jax.ShapeDtypeStruct((_NPK, 128), jnp.float32),
)


def kernel(x, edge_index, W1l, b1l, W1r, W2l, b2l, W2r):
  ei = edge_index.reshape(2, _NW, _NCHUNK, _C)
  w2l = jnp.zeros((_DHID, _F2), jnp.float32).at[:, :_DOUT].set(W2l.T)
  w2r = jnp.zeros((_DHID, _F2), jnp.float32).at[:, :_DOUT].set(W2r.T)
  zf1 = jnp.zeros((_RPT, _DHID), jnp.float32)
  zc = jnp.zeros((_RPT,), jnp.float32)
  zf2 = jnp.zeros((_RPT, _F2), jnp.float32)
  onesb = jnp.ones((_C,), jnp.float32)
  b1 = b1l.reshape(1, _DHID)
  b2 = jnp.zeros((1, _F2), jnp.float32).at[0, :_DOUT].set(b2l)
  b2p = jnp.tile(b2, (1, 8))
  lane = jnp.arange(128) // _F2
  msk = (lane[:, None] == lane[None, :]).astype(jnp.float32)
  w2r_bd = jnp.zeros((512, 128), jnp.float32)
  for i in range(8):
    w2r_bd = w2r_bd.at[64 * i:64 * (i + 1), 16 * i:16 * (i + 1)].set(w2r)
  b1p = jnp.tile(b1, (1, 2))
  lane64 = jnp.arange(128) // _DHID
  msk64 = (lane64[:, None] == lane64[None, :]).astype(jnp.float32)
  w2l_bd = jnp.zeros((128, 32), jnp.float32)
  for i in range(2):
    w2l_bd = w2l_bd.at[64 * i:64 * (i + 1), 16 * i:16 * (i + 1)].set(w2l)

  y1 = _mm_y1(x, W1l.T)
  s1p, rcnt = _pass1(y1, ei, zf1, zc, onesb)
  xr1 = _mm_xr(x, W1r.T)
  s1pk = s1p.reshape(_NC, _NP * _DHID // 128, 128)
  xrk = xr1.reshape(_NPK1, 128)
  y2k, hk = _mid(s1pk, xrk, b1p, msk64, w2l_bd)
  y2p = y2k.reshape(_N, _F2)
  s2p = _pass2(y2p, ei, zf2, rcnt)
  hp = hk.reshape(_N * _DHID // 512, 512)
  hrp = _mm_hr(hp, w2r_bd)
  outp = _fin(s2p.reshape(_NC, _NP * _F2 // 128, 128), hrp, b2p, msk)
  return outp.reshape(_N, _F2)[:, :_DOUT]


# speedup vs baseline: 1.1674x; 1.0002x over previous
import jax
import jax.numpy as jnp
from jax import lax
from jax.experimental import pallas as pl
from jax.experimental.pallas import tpu as pltpu
from jax.experimental.pallas import tpu_sc as plsc

_N = 10000
_E = 320000
_DIN = 128
_DHID = 64
_DOUT = 4

_NC = 2
_NS = 16
_NW = _NC * _NS
_EPW = _E // _NW
_C = 80
_NCHUNK = _EPW // _C
_B = 5
_NP = 10240
_RPT = _NP // _NS
_EP = 320
_F2 = 16

_mesh = plsc.VectorSubcoreMesh(core_axis_name="c", subcore_axis_name="s")
_sc_params = pltpu.CompilerParams(use_tc_tiling_on_sc=False)


def _stage_and_loop(y_hbm, sidx, didx, msg, acc, gsem, ssem, F,
                    extra_scatter=None):
  for b in range(_B):
    pltpu.async_copy(y_hbm.at[sidx.at[b]], msg.at[b], gsem[b])

  def drain_scatter(pb, j):
    pltpu.make_async_copy(msg.at[pb], acc.at[didx.at[j]], ssem[pb]).wait()
    if extra_scatter is not None:
      extra_scatter.drain(pb, j)

  def group(g, carry):
    for b in range(_B):
      j = g * _B + b
      pltpu.make_async_copy(y_hbm.at[sidx.at[j]], msg.at[b], gsem[b]).wait()
      pltpu.async_copy(msg.at[b], acc.at[didx.at[j]], ssem[b], add=True)
      if extra_scatter is not None:
        extra_scatter.issue(b, j)
      pb = (b - 1) % _B

      @pl.when(j >= 1)
      def _():
        drain_scatter(pb, j)

        @pl.when(j - 1 + _B < _NCHUNK)
        def _():
          pltpu.async_copy(y_hbm.at[sidx.at[j - 1 + _B]], msg.at[pb],
                           gsem[pb])
    return carry

  lax.fori_loop(0, _NCHUNK // _B, group, 0)
  drain_scatter(_B - 1, _NCHUNK - 1)


class _CountScatter:
  def __init__(self, onesv, cacc, didx, didx2, ssem):
    self.onesv, self.cacc, self.didx, self.didx2, self.ssem = (
        onesv, cacc, didx, didx2, ssem)

  def issue(self, b, j):
    pltpu.async_copy(self.onesv, self.cacc.at[self.didx.at[j]],
                     self.ssem[b], add=True)
    pltpu.async_copy(self.onesv, self.cacc.at[self.didx2.at[j]],
                     self.ssem[b], add=True)

  def drain(self, pb, j):
    pltpu.make_async_copy(self.onesv, self.cacc.at[self.didx.at[j]],
                          self.ssem[pb]).wait()
    pltpu.make_async_copy(self.onesv, self.cacc.at[self.didx2.at[j]],
                          self.ssem[pb]).wait()


def _make_pass1():
  out_type = (jax.ShapeDtypeStruct((_NC, _NP, _DHID), jnp.float32),
              jax.ShapeDtypeStruct((_NP,), jnp.float32))
  scratch = [
      pltpu.VMEM((_NCHUNK, _C), jnp.int32),
      pltpu.VMEM((_NCHUNK, _C), jnp.int32),
      pltpu.VMEM((_NCHUNK, _C), jnp.int32),
      pltpu.VMEM((_B, _C, _DHID), jnp.float32),
      pltpu.VMEM((_C,), jnp.float32),
      pltpu.VMEM((_EP, _DHID), jnp.float32),
      pltpu.VMEM((_EP,), jnp.float32),
      pltpu.VMEM((_EP,), jnp.float32),
      pltpu.VMEM_SHARED((_NP, _DHID), jnp.float32),
      pltpu.VMEM_SHARED((_NP,), jnp.float32),
  ] + [pltpu.SemaphoreType.DMA] * (2 * _B)

  def body(y_hbm, ei_hbm, zf_hbm, zc_hbm, ones_hbm, out_hbm, rcnt_hbm,
           sidx, didx, didx2, msg, onesv, eb, cb, rb, acc, cacc, *sems):
    gsem, ssem = sems[:_B], sems[_B:]
    c = lax.axis_index("c")
    s = lax.axis_index("s")
    wid = c * _NS + s
    wid2 = (1 - c) * _NS + s
    rows = pl.ds(s * _RPT, _RPT)
    pltpu.sync_copy(zf_hbm, acc.at[rows])
    pltpu.sync_copy(zc_hbm, cacc.at[rows])
    pltpu.sync_copy(ei_hbm.at[0].at[wid], sidx)
    pltpu.sync_copy(ei_hbm.at[1].at[wid], didx)
    pltpu.sync_copy(ei_hbm.at[1].at[wid2], didx2)
    pltpu.sync_copy(ones_hbm, onesv)
    plsc.subcore_barrier()

    cs = _CountScatter(onesv, cacc, didx, didx2, ssem)
    _stage_and_loop(y_hbm, sidx, didx, msg, acc, gsem, ssem, _DHID,
                    extra_scatter=cs)
    plsc.subcore_barrier()

    for half in range(2):
      rbase = s * _RPT + half * _EP
      strip = pl.ds(rbase, _EP)
      pltpu.sync_copy(acc.at[strip], eb)
      pltpu.sync_copy(cacc.at[strip], cb)

      for k in range(_EP // 16):
        lanes = pl.ds(16 * k, 16)
        rb[lanes] = 1.0 / jnp.maximum(cb[lanes], 1.0)

      def egroup(g, carry):
        cvec = rb[pl.ds(g * 16, 16)]
        for l in range(16):
          rr = g * 16 + l
          rv = cvec[l]
          for k in range(_DHID // 16):
            col = pl.ds(16 * k, 16)
            eb[rr, col] = eb[rr, col] * rv
        return carry

      lax.fori_loop(0, _EP // 16, egroup, 0)
      pltpu.sync_copy(eb, out_hbm.at[c].at[strip])

      @pl.when(c == 0)
      def _():
        pltpu.sync_copy(rb, rcnt_hbm.at[strip])

  return pl.kernel(body, mesh=_mesh, out_type=out_type,
                   scratch_types=scratch, compiler_params=_sc_params)


def _make_pass2():
  out_type = jax.ShapeDtypeStruct((_NC, _NP, _F2), jnp.float32)
  scratch = [
      pltpu.VMEM((_NCHUNK, _C), jnp.int32),
      pltpu.VMEM((_NCHUNK, _C), jnp.int32),
      pltpu.VMEM((_B, _C, _F2), jnp.float32),
      pltpu.VMEM((_RPT,), jnp.float32),
      pltpu.VMEM((_EP, _F2), jnp.float32),
      pltpu.VMEM_SHARED((_NP, _F2), jnp.float32),
  ] + [pltpu.SemaphoreType.DMA] * (2 * _B)

  def body(y_hbm, ei_hbm, zf_hbm, rcnt_hbm, out_hbm,
           sidx, didx, msg, rstage, eb, acc, *sems):
    gsem, ssem = sems[:_B], sems[_B:]
    c = lax.axis_index("c")
    s = lax.axis_index("s")
    wid = c * _NS + s
    rows = pl.ds(s * _RPT, _RPT)
    pltpu.sync_copy(zf_hbm, acc.at[rows])
    pltpu.sync_copy(ei_hbm.at[0].at[wid], sidx)
    pltpu.sync_copy(ei_hbm.at[1].at[wid], didx)
    pltpu.sync_copy(rcnt_hbm.at[rows], rstage)
    plsc.subcore_barrier()

    _stage_and_loop(y_hbm, sidx, didx, msg, acc, gsem, ssem, _F2)
    plsc.subcore_barrier()

    for half in range(2):
      strip = pl.ds(s * _RPT + half * _EP, _EP)
      pltpu.sync_copy(acc.at[strip], eb)

      def egroup(g, carry):
        cvec = rstage[pl.ds(half * _EP + g * 16, 16)]
        for l in range(16):
          rr = g * 16 + l
          eb[rr] = eb[rr] * cvec[l]
        return carry

      lax.fori_loop(0, _EP // 16, egroup, 0)
      pltpu.sync_copy(eb, out_hbm.at[c].at[strip])

  return pl.kernel(body, mesh=_mesh, out_type=out_type,
                   scratch_types=scratch, compiler_params=_sc_params)


_pass1 = _make_pass1()
_pass2 = _make_pass2()


def _mm_body(x_ref, w_ref, y_ref):
  y_ref[...] = jnp.dot(x_ref[...], w_ref[...],
                       preferred_element_type=jnp.float32)


def _make_mm(n, k, m):
  return pl.pallas_call(
      _mm_body, out_shape=jax.ShapeDtypeStruct((n, m), jnp.float32))


_mm_y1 = _make_mm(_N // 2, 2 * _DIN, 128)
_mm_xr = _make_mm(_N // 2, 2 * _DIN, 128)
_mm_hr = _make_mm(_N * _DHID // 512, 512, 128)


_NPK1 = _N * _DHID // 128


def _mid_body(s1p_ref, xr_ref, b1_ref, msk_ref, w2_ref, y2_ref, h_ref):
  sp = s1p_ref[...]
  o = sp[0, :_NPK1] + sp[1, :_NPK1] + b1_ref[...] + xr_ref[...]
  ss = jnp.dot(o * o, msk_ref[...], preferred_element_type=jnp.float32)
  h = jnp.maximum(o / jnp.maximum(jnp.sqrt(ss), 1e-12), 0.0)
  h_ref[...] = h
  y2_ref[...] = jnp.dot(h, w2_ref[...], preferred_element_type=jnp.float32)


_mid = pl.pallas_call(
    _mid_body,
    out_shape=(jax.ShapeDtypeStruct((_NPK1, 32), jnp.float32),
               jax.ShapeDtypeStruct((_NPK1, 128), jnp.float32)),
)


_NPK = _N * _F2 // 128


def _fin_body(s2p_ref, hrp_ref, b2p_ref, msk_ref, o_ref):
  sp = s2p_ref[...]
  o = sp[0, :_NPK] + sp[1, :_NPK] + b2p_ref[...] + hrp_ref[...]
  ss = jnp.dot(o * o, msk_ref[...], preferred_element_type=jnp.float32)
  o_ref[...] = o / jnp.maximum(jnp.sqrt(ss), 1e-12)


_fin = pl.pallas_call(
    _fin_body,
    out_shape=jax.ShapeDtypeStruct((_NPK, 128), jnp.float32),
)


def kernel(x, edge_index, W1l, b1l, W1r, W2l, b2l, W2r):
  ei = edge_index.reshape(2, _NW, _NCHUNK, _C)
  w2l = jnp.zeros((_DHID, _F2), jnp.float32).at[:, :_DOUT].set(W2l.T)
  w2r = jnp.zeros((_DHID, _F2), jnp.float32).at[:, :_DOUT].set(W2r.T)
  zf1 = jnp.zeros((_RPT, _DHID), jnp.float32)
  zc = jnp.zeros((_RPT,), jnp.float32)
  zf2 = jnp.zeros((_RPT, _F2), jnp.float32)
  onesb = jnp.ones((_C,), jnp.float32)
  b1 = b1l.reshape(1, _DHID)
  b2 = jnp.zeros((1, _F2), jnp.float32).at[0, :_DOUT].set(b2l)
  b2p = jnp.tile(b2, (1, 8))
  lane = jnp.arange(128) // _F2
  msk = (lane[:, None] == lane[None, :]).astype(jnp.float32)
  w2r_bd = jnp.zeros((512, 128), jnp.float32)
  for i in range(8):
    w2r_bd = w2r_bd.at[64 * i:64 * (i + 1), 16 * i:16 * (i + 1)].set(w2r)
  b1p = jnp.tile(b1, (1, 2))
  lane64 = jnp.arange(128) // _DHID
  msk64 = (lane64[:, None] == lane64[None, :]).astype(jnp.float32)
  w2l_bd = jnp.zeros((128, 32), jnp.float32)
  for i in range(2):
    w2l_bd = w2l_bd.at[64 * i:64 * (i + 1), 16 * i:16 * (i + 1)].set(w2l)
  w1l_bd = jnp.zeros((2 * _DIN, 128), jnp.float32)
  w1r_bd = jnp.zeros((2 * _DIN, 128), jnp.float32)
  for i in range(2):
    w1l_bd = w1l_bd.at[_DIN * i:_DIN * (i + 1),
                       _DHID * i:_DHID * (i + 1)].set(W1l.T)
    w1r_bd = w1r_bd.at[_DIN * i:_DIN * (i + 1),
                       _DHID * i:_DHID * (i + 1)].set(W1r.T)
  x2 = x.reshape(_N // 2, 2 * _DIN)

  y1 = _mm_y1(x2, w1l_bd).reshape(_N, _DHID)
  s1p, rcnt = _pass1(y1, ei, zf1, zc, onesb)
  xrk = _mm_xr(x2, w1r_bd)
  s1pk = s1p.reshape(_NC, _NP * _DHID // 128, 128)
  y2k, hk = _mid(s1pk, xrk, b1p, msk64, w2l_bd)
  y2p = y2k.reshape(_N, _F2)
  s2p = _pass2(y2p, ei, zf2, rcnt)
  hp = hk.reshape(_N * _DHID // 512, 512)
  hrp = _mm_hr(hp, w2r_bd)
  outp = _fin(s2p.reshape(_NC, _NP * _F2 // 128, 128), hrp, b2p, msk)
  return outp.reshape(_N, _F2)[:, :_DOUT]
